# baseline (speedup 1.0000x reference)
import jax
import jax.numpy as jnp
from jax.experimental import pallas as pl

E = 2048
N = 1024
OUT = 2 * E + N
DT = 1e-06

R = 1024
NB = (N + 2 * E) // R
B1 = N // R
B2 = (N + E) // R


def _band_kernel(m_ref, p_ref, k_ref, s_ref, out_ref):
    i = pl.program_id(0)

    @pl.when(i == 0)
    def _kcl():
        out_ref[:, 0:E] = m_ref[...]
        out_ref[:, E:] = jnp.zeros((R, OUT - E), jnp.float32)

    def _kvl(half):
        e0 = half * R
        rows = jax.lax.broadcasted_iota(jnp.int32, (R, E), 0)
        cols = jax.lax.broadcasted_iota(jnp.int32, (R, E), 1)
        out_ref[:, 0:E] = jnp.zeros((R, E), jnp.float32)
        out_ref[:, E:2 * E] = jnp.where(cols == rows + e0, 1.0, 0.0)
        out_ref[:, 2 * E:] = -m_ref[:, e0:e0 + R].T

    @pl.when(i == B1)
    def _kvl0():
        _kvl(0)

    @pl.when(i == B1 + 1)
    def _kvl1():
        _kvl(1)

    @pl.when(i >= B2)
    def _el():
        e0 = (i - B2) * R
        params = p_ref[...]
        kinds = k_ref[...]
        sw_on = s_ref[...] > 0.0
        z = jnp.where(kinds == 0, -params,
            jnp.where(kinds == 4, -DT / params,
            jnp.where(kinds == 5, 1.0,
            jnp.where(kinds == 2, 1.0,
            jnp.where(jnp.logical_and(kinds == 3, jnp.logical_not(sw_on)),
                      1.0, 0.0)))))
        y = jnp.where(kinds == 0, 1.0,
            jnp.where(kinds == 4, 1.0,
            jnp.where(kinds == 5, -DT / params,
            jnp.where(kinds == 1, 1.0,
            jnp.where(jnp.logical_and(kinds == 3, sw_on), 1.0, 0.0)))))
        rows = jax.lax.broadcasted_iota(jnp.int32, (R, E), 0)
        cols = jax.lax.broadcasted_iota(jnp.int32, (R, E), 1)
        diag = cols == rows + e0
        out_ref[:, 0:E] = jnp.where(diag, z, 0.0)
        out_ref[:, E:2 * E] = jnp.where(diag, y, 0.0)
        out_ref[:, 2 * E:] = jnp.zeros((R, N), jnp.float32)


def kernel(M, params, sw_params, kinds, time):
    swcol = sw_params[:, time]
    p2 = params.reshape(1, E).astype(jnp.float32)
    k2 = kinds.reshape(1, E).astype(jnp.int32)
    s2 = swcol.reshape(1, E).astype(jnp.float32)

    out = pl.pallas_call(
        _band_kernel,
        grid=(NB,),
        in_specs=[
            pl.BlockSpec((N, E), lambda i: (0, 0)),
            pl.BlockSpec((1, E), lambda i: (0, 0)),
            pl.BlockSpec((1, E), lambda i: (0, 0)),
            pl.BlockSpec((1, E), lambda i: (0, 0)),
        ],
        out_specs=pl.BlockSpec((R, OUT), lambda i: (i, 0)),
        out_shape=jax.ShapeDtypeStruct((OUT, OUT), jnp.float32),
    )(M, p2, k2, s2)
    return out


# speedup vs baseline: 1.0340x; 1.0340x over previous
import jax
import jax.numpy as jnp
from jax.experimental import pallas as pl

E = 2048
N = 1024
OUT = 2 * E + N
DT = 1e-06

R = 512
NB = OUT // R
B1 = N // R
B2 = (N + E) // R


def _band_kernel(m_ref, p_ref, k_ref, s_ref, out_ref):
    i = pl.program_id(0)

    def _kcl(r0):
        out_ref[:, 0:E] = m_ref[r0:r0 + R, :]
        out_ref[:, E:] = jnp.zeros((R, OUT - E), jnp.float32)

    def _kvl(e0):
        rows = jax.lax.broadcasted_iota(jnp.int32, (R, E), 0)
        cols = jax.lax.broadcasted_iota(jnp.int32, (R, E), 1)
        out_ref[:, 0:E] = jnp.zeros((R, E), jnp.float32)
        out_ref[:, E:2 * E] = jnp.where(cols == rows + e0, 1.0, 0.0)
        out_ref[:, 2 * E:] = -m_ref[:, e0:e0 + R].T

    for b in range(B1):
        pl.when(i == b)(lambda b=b: _kcl(b * R))
    for b in range(B1, B2):
        pl.when(i == b)(lambda b=b: _kvl((b - B1) * R))

    @pl.when(i >= B2)
    def _el():
        e0 = (i - B2) * R
        params = p_ref[...]
        kinds = k_ref[...]
        sw_on = s_ref[...] > 0.0
        z = jnp.where(kinds == 0, -params,
            jnp.where(kinds == 4, -DT / params,
            jnp.where(kinds == 5, 1.0,
            jnp.where(kinds == 2, 1.0,
            jnp.where(jnp.logical_and(kinds == 3, jnp.logical_not(sw_on)),
                      1.0, 0.0)))))
        y = jnp.where(kinds == 0, 1.0,
            jnp.where(kinds == 4, 1.0,
            jnp.where(kinds == 5, -DT / params,
            jnp.where(kinds == 1, 1.0,
            jnp.where(jnp.logical_and(kinds == 3, sw_on), 1.0, 0.0)))))
        rows = jax.lax.broadcasted_iota(jnp.int32, (R, E), 0)
        cols = jax.lax.broadcasted_iota(jnp.int32, (R, E), 1)
        diag = cols == rows + e0
        out_ref[:, 0:E] = jnp.where(diag, z, 0.0)
        out_ref[:, E:2 * E] = jnp.where(diag, y, 0.0)
        out_ref[:, 2 * E:] = jnp.zeros((R, N), jnp.float32)


def kernel(M, params, sw_params, kinds, time):
    swcol = sw_params[:, time]
    p2 = params.reshape(1, E).astype(jnp.float32)
    k2 = kinds.reshape(1, E).astype(jnp.int32)
    s2 = swcol.reshape(1, E).astype(jnp.float32)

    out = pl.pallas_call(
        _band_kernel,
        grid=(NB,),
        in_specs=[
            pl.BlockSpec((N, E), lambda i: (0, 0)),
            pl.BlockSpec((1, E), lambda i: (0, 0)),
            pl.BlockSpec((1, E), lambda i: (0, 0)),
            pl.BlockSpec((1, E), lambda i: (0, 0)),
        ],
        out_specs=pl.BlockSpec((R, OUT), lambda i: (i, 0)),
        out_shape=jax.ShapeDtypeStruct((OUT, OUT), jnp.float32),
    )(M, p2, k2, s2)
    return out


# speedup vs baseline: 1.1010x; 1.0648x over previous
import jax
import jax.numpy as jnp
from jax.experimental import pallas as pl
from jax.experimental.pallas import tpu as pltpu

E = 2048
N = 1024
OUT = 2 * E + N
DT = 1e-06

R = 512
NB = OUT // R
N_KCL = N // R
N_KVL = E // R
N_EL = E // R


def _band_kernel(m_hbm, p_ref, k_ref, s_ref, out_ref, m_vmem, sem):
    s = pl.program_id(0)

    @pl.when(s == 0)
    def _start_m_copy():
        pltpu.make_async_copy(m_hbm, m_vmem, sem).start()

    @pl.when(s == N_EL)
    def _wait_m_copy():
        pltpu.make_async_copy(m_hbm, m_vmem, sem).wait()

    @pl.when(s < N_EL)
    def _el():
        e0 = s * R
        params = p_ref[...]
        kinds = k_ref[...]
        sw_on = s_ref[...] > 0.0
        z = jnp.where(kinds == 0, -params,
            jnp.where(kinds == 4, -DT / params,
            jnp.where(kinds == 5, 1.0,
            jnp.where(kinds == 2, 1.0,
            jnp.where(jnp.logical_and(kinds == 3, jnp.logical_not(sw_on)),
                      1.0, 0.0)))))
        y = jnp.where(kinds == 0, 1.0,
            jnp.where(kinds == 4, 1.0,
            jnp.where(kinds == 5, -DT / params,
            jnp.where(kinds == 1, 1.0,
            jnp.where(jnp.logical_and(kinds == 3, sw_on), 1.0, 0.0)))))
        rows = jax.lax.broadcasted_iota(jnp.int32, (R, E), 0)
        cols = jax.lax.broadcasted_iota(jnp.int32, (R, E), 1)
        diag = cols == rows + e0
        out_ref[:, 0:E] = jnp.where(diag, z, 0.0)
        out_ref[:, E:2 * E] = jnp.where(diag, y, 0.0)
        out_ref[:, 2 * E:] = jnp.zeros((R, N), jnp.float32)

    def _kvl(e0):
        rows = jax.lax.broadcasted_iota(jnp.int32, (R, E), 0)
        cols = jax.lax.broadcasted_iota(jnp.int32, (R, E), 1)
        out_ref[:, 0:E] = jnp.zeros((R, E), jnp.float32)
        out_ref[:, E:2 * E] = jnp.where(cols == rows + e0, 1.0, 0.0)
        out_ref[:, 2 * E:] = -m_vmem[:, e0:e0 + R].T

    def _kcl(r0):
        out_ref[:, 0:E] = m_vmem[r0:r0 + R, :]
        out_ref[:, E:] = jnp.zeros((R, OUT - E), jnp.float32)

    for b in range(N_KVL):
        pl.when(s == N_EL + b)(lambda b=b: _kvl(b * R))
    for b in range(N_KCL):
        pl.when(s == N_EL + N_KVL + b)(lambda b=b: _kcl(b * R))


def _out_band(s):
    return jnp.where(s < N_EL, s + N_KCL + N_KVL,
           jnp.where(s < N_EL + N_KVL, s - N_EL + N_KCL,
                     s - N_EL - N_KVL))


def kernel(M, params, sw_params, kinds, time):
    swcol = sw_params[:, time]
    p2 = params.reshape(1, E).astype(jnp.float32)
    k2 = kinds.reshape(1, E).astype(jnp.int32)
    s2 = swcol.reshape(1, E).astype(jnp.float32)

    out = pl.pallas_call(
        _band_kernel,
        grid=(NB,),
        in_specs=[
            pl.BlockSpec(memory_space=pl.ANY),
            pl.BlockSpec((1, E), lambda i: (0, 0)),
            pl.BlockSpec((1, E), lambda i: (0, 0)),
            pl.BlockSpec((1, E), lambda i: (0, 0)),
        ],
        out_specs=pl.BlockSpec((R, OUT), lambda i: (_out_band(i), 0)),
        out_shape=jax.ShapeDtypeStruct((OUT, OUT), jnp.float32),
        scratch_shapes=[
            pltpu.VMEM((N, E), jnp.float32),
            pltpu.SemaphoreType.DMA,
        ],
    )(M, p2, k2, s2)
    return out
